# baseline (speedup 1.0000x reference)
import functools

import jax
import jax.numpy as jnp
from jax import lax
from jax.experimental import pallas as pl
from jax.experimental.pallas import tpu as pltpu
from jax.experimental.pallas import tpu_sc as plsc

_NC = 2
_NS = 16
_NW = _NC * _NS
_K = 8
_B = 4096
_D = 64
_S = 50
_NUNITS = _S * (_D // _K)
_NROUNDS = (_NUNITS + _NW - 1) // _NW


def _sc_add(t_hbm, kp_hbm, out_hbm, kp_v, buf0, buf1, in_s0, in_s1, out_s0, out_s1):
    wid = lax.axis_index("s") * _NC + lax.axis_index("c")
    bufs = (buf0, buf1)
    in_sems = (in_s0, in_s1)
    out_sems = (out_s0, out_s1)

    pltpu.sync_copy(kp_hbm, kp_v)

    def unit_slice(u):
        s = u // (_D // _K)
        d0 = (u % (_D // _K)) * _K
        return s, d0

    def start_in(u, p):
        s, d0 = unit_slice(u)
        pltpu.async_copy(t_hbm.at[s, pl.ds(d0, _K)], bufs[p], in_sems[p])

    def wait_in(p):
        pltpu.make_async_copy(t_hbm.at[0, pl.ds(0, _K)], bufs[p], in_sems[p]).wait()

    def start_out(u, p):
        s, d0 = unit_slice(u)
        pltpu.async_copy(bufs[p], out_hbm.at[s, pl.ds(d0, _K)], out_sems[p])

    def wait_out(p):
        pltpu.make_async_copy(t_hbm.at[0, pl.ds(0, _K)], bufs[p], out_sems[p]).wait()

    def compute(u, p):
        _, d0 = unit_slice(u)
        buf = bufs[p]
        for dd in range(_K):
            kpv = plsc.load_gather(kp_v, [jnp.full((16,), d0 + dd, jnp.int32)])

            def body(i, _, dd=dd, kpv=kpv, buf=buf):
                buf[dd, pl.ds(i * 16, 16)] = buf[dd, pl.ds(i * 16, 16)] + kpv
                return 0

            lax.fori_loop(0, _B // 16, body, 0)

    @pl.when(wid < _NUNITS)
    def _():
        start_in(wid, 0)

    for k in range(_NROUNDS):
        p = k & 1
        q = 1 - p
        u = wid + _NW * k
        if k + 1 < _NROUNDS:
            u2 = wid + _NW * (k + 1)

            if k >= 1:
                @pl.when(u2 < _NUNITS)
                def _(q=q):
                    wait_out(q)

            @pl.when(u2 < _NUNITS)
            def _(u2=u2, q=q):
                start_in(u2, q)

        @pl.when(u < _NUNITS)
        def _(u=u, p=p):
            wait_in(p)
            compute(u, p)
            start_out(u, p)

    for k in (_NROUNDS - 2, _NROUNDS - 1):
        @pl.when(wid + _NW * k < _NUNITS)
        def _(k=k):
            wait_out(k & 1)


def kernel(src, mask, pos_embed, tgt, tgt_mask, class_feature, kp_template):
    S, B, D = tgt.shape
    t_t = jnp.swapaxes(tgt, 1, 2)

    mesh = plsc.VectorSubcoreMesh(core_axis_name="c", subcore_axis_name="s")
    sc_fn = functools.partial(
        pl.kernel,
        mesh=mesh,
        out_type=jax.ShapeDtypeStruct((S, D, B), jnp.float32),
        compiler_params=pltpu.CompilerParams(needs_layout_passes=False),
        scratch_types=[
            pltpu.VMEM((D,), jnp.float32),
            pltpu.VMEM((_K, B), jnp.float32),
            pltpu.VMEM((_K, B), jnp.float32),
            pltpu.SemaphoreType.DMA,
            pltpu.SemaphoreType.DMA,
            pltpu.SemaphoreType.DMA,
            pltpu.SemaphoreType.DMA,
        ],
    )(_sc_add)
    out_t = sc_fn(t_t, kp_template)
    return jnp.swapaxes(out_t, 1, 2)


# speedup vs baseline: 5.7275x; 5.7275x over previous
import jax
import jax.numpy as jnp
from jax.experimental import pallas as pl
from jax.experimental.pallas import tpu as pltpu


def _add_body(t_ref, k_ref, o_ref):
    kp_col = k_ref[...].T
    o_ref[...] = t_ref[...] + kp_col[None]


def kernel(src, mask, pos_embed, tgt, tgt_mask, class_feature, kp_template):
    S, B, D = tgt.shape
    t_t = jnp.swapaxes(tgt, 1, 2)
    kp2 = kp_template.reshape(1, D)

    BLK_S = 15
    grid = ((S + BLK_S - 1) // BLK_S,)
    out_t = pl.pallas_call(
        _add_body,
        grid=grid,
        compiler_params=pltpu.CompilerParams(dimension_semantics=("parallel",), vmem_limit_bytes=63 * 1024 * 1024),
        in_specs=[
            pl.BlockSpec((BLK_S, D, B), lambda i: (i, 0, 0)),
            pl.BlockSpec((1, D), lambda i: (0, 0)),
        ],
        out_specs=pl.BlockSpec((BLK_S, D, B), lambda i: (i, 0, 0)),
        out_shape=jax.ShapeDtypeStruct((S, D, B), tgt.dtype),
    )(t_t, kp2)
    return jnp.swapaxes(out_t, 1, 2)
